# baseline (speedup 1.0000x reference)
import functools
import math

import jax
import jax.numpy as jnp
from jax import lax
from jax.experimental import pallas as pl
from jax.experimental.pallas import tpu as pltpu
from jax.experimental.pallas import tpu_sc as plsc

D_MODEL = 64
SCALE = math.sqrt(D_MODEL)
LANES = 16

NUM_CORES = 2
NUM_SUBCORES = 16
NW = NUM_CORES * NUM_SUBCORES

B = 4096 * 200
SUB = 128
B_PER_W = B // NW
CHUNK_ROWS = 4
CHUNK = CHUNK_ROWS * SUB
N_CHUNKS = B_PER_W // CHUNK
ROWS_PER_W = B_PER_W // SUB

_mesh = plsc.VectorSubcoreMesh(core_axis_name="c", subcore_axis_name="s")


@functools.partial(
    pl.kernel,
    out_type=jax.ShapeDtypeStruct((B, D_MODEL), jnp.float32),
    mesh=_mesh,
    scratch_types=[
        pltpu.VMEM((CHUNK_ROWS, SUB), jnp.int32),
        pltpu.VMEM((CHUNK, D_MODEL), jnp.float32),
        pltpu.SemaphoreType.DMA,
    ],
    compiler_params=pltpu.CompilerParams(use_tc_tiling_on_sc=False),
)
def _emb_lookup(idx_hbm, table_hbm, out_hbm, idx_v, rows_v, sem):
    wid = lax.axis_index("s") * NUM_CORES + lax.axis_index("c")
    row_base = wid * ROWS_PER_W

    def chunk_body(g, carry):
        roff = row_base + g * CHUNK_ROWS
        off = roff * SUB
        pltpu.sync_copy(idx_hbm.at[pl.ds(roff, CHUNK_ROWS)], idx_v)
        copies = []
        for j in range(CHUNK_ROWS):
            copies.append(
                pltpu.async_copy(
                    table_hbm.at[idx_v.at[j]],
                    rows_v.at[pl.ds(j * SUB, SUB)],
                    sem,
                )
            )
        for c in copies:
            c.wait()

        def row_body(r, c2):
            for q in range(D_MODEL // LANES):
                sl = pl.ds(q * LANES, LANES)
                rows_v[r, sl] = rows_v[r, sl] * SCALE
            return c2

        lax.fori_loop(0, CHUNK, row_body, 0)

        pltpu.sync_copy(rows_v, out_hbm.at[pl.ds(off, CHUNK)])
        return carry

    lax.fori_loop(0, N_CHUNKS, chunk_body, 0)


def kernel(x, table):
    idx = x.reshape(-1).astype(jnp.int32).reshape(B // SUB, SUB)
    out = _emb_lookup(idx, table)
    return out.reshape(x.shape + (D_MODEL,))


# speedup vs baseline: 1.1358x; 1.1358x over previous
import functools
import math

import jax
import jax.numpy as jnp
from jax import lax
from jax.experimental import pallas as pl
from jax.experimental.pallas import tpu as pltpu
from jax.experimental.pallas import tpu_sc as plsc

D_MODEL = 64
SCALE = math.sqrt(D_MODEL)
LANES = 16

NUM_CORES = 2
NUM_SUBCORES = 16
NW = NUM_CORES * NUM_SUBCORES

B = 4096 * 200
SUB = 128
B_PER_W = B // NW
ROWS_PER_W = B_PER_W // SUB
CHUNK_ROWS = 2
CHUNK = CHUNK_ROWS * SUB
N_CHUNKS = B_PER_W // CHUNK
NBUF = 4
FIRE_AHEAD = NBUF - 2

_mesh = plsc.VectorSubcoreMesh(core_axis_name="c", subcore_axis_name="s")


@functools.partial(
    pl.kernel,
    out_type=jax.ShapeDtypeStruct((B, D_MODEL), jnp.float32),
    mesh=_mesh,
    scratch_types=[
        pltpu.VMEM((ROWS_PER_W, SUB), jnp.int32),
        [pltpu.VMEM((CHUNK, D_MODEL), jnp.float32) for _ in range(NBUF)],
        [pltpu.SemaphoreType.DMA for _ in range(NBUF)],
        [pltpu.SemaphoreType.DMA for _ in range(NBUF)],
    ],
    compiler_params=pltpu.CompilerParams(use_tc_tiling_on_sc=False),
)
def _emb_lookup(idx_hbm, table_hbm, out_hbm, idx_v, rows, sem_g, sem_s):
    wid = lax.axis_index("s") * NUM_CORES + lax.axis_index("c")
    base = wid * B_PER_W
    row_base = wid * ROWS_PER_W

    pltpu.sync_copy(idx_hbm.at[pl.ds(row_base, ROWS_PER_W)], idx_v)

    def fire_gathers(f, b):
        for j in range(CHUNK_ROWS):
            pltpu.async_copy(
                table_hbm.at[idx_v.at[f * CHUNK_ROWS + j]],
                rows[b].at[pl.ds(j * SUB, SUB)],
                sem_g[b],
            )

    def wait_gathers(b):
        for j in range(CHUNK_ROWS):
            pltpu.make_async_copy(
                table_hbm.at[idx_v.at[j]],
                rows[b].at[pl.ds(j * SUB, SUB)],
                sem_g[b],
            ).wait()

    def wait_store(b):
        pltpu.make_async_copy(
            rows[b], out_hbm.at[pl.ds(0, CHUNK)], sem_s[b]
        ).wait()

    for f in range(FIRE_AHEAD):
        fire_gathers(f, f)

    def outer(g0, carry):
        for b in range(NBUF):
            g = g0 * NBUF + b
            f = g + FIRE_AHEAD
            fb = (b + FIRE_AHEAD) % NBUF

            @pl.when((f < N_CHUNKS) & (g >= FIRE_AHEAD))
            def _():
                wait_store(fb)

            @pl.when(f < N_CHUNKS)
            def _():
                fire_gathers(f, fb)

            wait_gathers(b)

            @plsc.parallel_loop(0, CHUNK, unroll=8)
            def _(r):
                for q in range(D_MODEL // LANES):
                    sl = pl.ds(q * LANES, LANES)
                    rows[b][r, sl] = rows[b][r, sl] * SCALE

            pltpu.async_copy(
                rows[b], out_hbm.at[pl.ds(base + g * CHUNK, CHUNK)], sem_s[b]
            )
        return carry

    lax.fori_loop(0, N_CHUNKS // NBUF, outer, 0)

    for b in range(NBUF):
        wait_store(b)


def kernel(x, table):
    idx = x.reshape(-1).astype(jnp.int32).reshape(B // SUB, SUB)
    out = _emb_lookup(idx, table)
    return out.reshape(x.shape + (D_MODEL,))
